# Initial kernel scaffold; baseline (speedup 1.0000x reference)
import jax
import jax.numpy as jnp
from jax.experimental import pallas as pl


def kernel(indices, table):
    raise NotImplementedError("write your pallas kernel here")


# speedup vs baseline: 1.4908x; 1.4908x over previous
import functools

import jax
import jax.numpy as jnp
from jax import lax
from jax.experimental import pallas as pl
from jax.experimental.pallas import tpu as pltpu
from jax.experimental.pallas import tpu_sc as plsc

BATCH = 16384
HIST = 20
EMBED_DIM = 32
NUM_ROWS = BATCH * HIST
NC, NS = 2, 16
NW = NC * NS
ROWS_PER_W = NUM_ROWS // NW
CHUNK = 1024
N_CHUNKS = ROWS_PER_W // CHUNK


def _gather_body(idx_hbm, table_hbm, out_hbm, idx_v, rows_v, sem):
    wid = lax.axis_index("s") * NC + lax.axis_index("c")
    base = wid * ROWS_PER_W

    def chunk_step(g, _):
        off = base + g * CHUNK
        pltpu.sync_copy(idx_hbm.at[pl.ds(off, CHUNK)], idx_v)
        pltpu.async_copy(table_hbm.at[idx_v], rows_v, sem).wait()
        pltpu.sync_copy(rows_v, out_hbm.at[pl.ds(off, CHUNK)])
        return _

    lax.fori_loop(0, N_CHUNKS, chunk_step, 0)


@functools.partial(jax.jit, static_argnames=())
def kernel(indices, table):
    idx_flat = indices.reshape(NUM_ROWS).astype(jnp.int32)
    mesh = plsc.VectorSubcoreMesh(
        core_axis_name="c", subcore_axis_name="s",
        num_cores=NC, num_subcores=NS,
    )
    run = pl.kernel(
        _gather_body,
        out_type=jax.ShapeDtypeStruct((NUM_ROWS, EMBED_DIM), jnp.float32),
        mesh=mesh,
        scratch_types=[
            pltpu.VMEM((CHUNK,), jnp.int32),
            pltpu.VMEM((CHUNK, EMBED_DIM), jnp.float32),
            pltpu.SemaphoreType.DMA,
        ],
        compiler_params=pltpu.CompilerParams(use_tc_tiling_on_sc=False),
    )
    out = run(idx_flat, table)
    return out.reshape(BATCH, HIST, EMBED_DIM)


# speedup vs baseline: 1.5135x; 1.0152x over previous
import functools

import jax
import jax.numpy as jnp
from jax import lax
from jax.experimental import pallas as pl
from jax.experimental.pallas import tpu as pltpu
from jax.experimental.pallas import tpu_sc as plsc

BATCH = 16384
HIST = 20
EMBED_DIM = 32
NUM_ROWS = BATCH * HIST
NC, NS = 2, 16
NW = NC * NS
ROWS_PER_W = NUM_ROWS // NW
CHUNK = 1024
N_CHUNKS = ROWS_PER_W // CHUNK


def _gather_body(idx_hbm, table_hbm, out_hbm, idx_v, rows0, rows1, sem_g,
                 sem_o):
    wid = lax.axis_index("s") * NC + lax.axis_index("c")
    base = wid * ROWS_PER_W

    pltpu.sync_copy(idx_hbm.at[pl.ds(base, ROWS_PER_W)], idx_v)

    bufs = (rows0, rows1)
    gathers = [None] * N_CHUNKS
    writes = [None] * N_CHUNKS
    for g in range(N_CHUNKS):
        buf = bufs[g % 2]
        if g >= 2:
            writes[g - 2].wait()
        gathers[g] = pltpu.async_copy(
            table_hbm.at[idx_v.at[pl.ds(g * CHUNK, CHUNK)]], buf, sem_g)
        if g >= 1:
            gathers[g - 1].wait()
            writes[g - 1] = pltpu.async_copy(
                bufs[(g - 1) % 2],
                out_hbm.at[pl.ds(base + (g - 1) * CHUNK, CHUNK)], sem_o)
    gathers[N_CHUNKS - 1].wait()
    writes[N_CHUNKS - 1] = pltpu.async_copy(
        bufs[(N_CHUNKS - 1) % 2],
        out_hbm.at[pl.ds(base + (N_CHUNKS - 1) * CHUNK, CHUNK)], sem_o)
    writes[N_CHUNKS - 2].wait()
    writes[N_CHUNKS - 1].wait()


@functools.partial(jax.jit, static_argnames=())
def kernel(indices, table):
    idx_flat = indices.reshape(NUM_ROWS).astype(jnp.int32)
    mesh = plsc.VectorSubcoreMesh(
        core_axis_name="c", subcore_axis_name="s",
        num_cores=NC, num_subcores=NS,
    )
    run = pl.kernel(
        _gather_body,
        out_type=jax.ShapeDtypeStruct((NUM_ROWS, EMBED_DIM), jnp.float32),
        mesh=mesh,
        scratch_types=[
            pltpu.VMEM((ROWS_PER_W,), jnp.int32),
            pltpu.VMEM((CHUNK, EMBED_DIM), jnp.float32),
            pltpu.VMEM((CHUNK, EMBED_DIM), jnp.float32),
            pltpu.SemaphoreType.DMA,
            pltpu.SemaphoreType.DMA,
        ],
        compiler_params=pltpu.CompilerParams(use_tc_tiling_on_sc=False),
    )
    out = run(idx_flat, table)
    return out.reshape(BATCH, HIST, EMBED_DIM)
